# baseline (speedup 1.0000x reference)
import functools

import jax
import jax.numpy as jnp
from jax import lax
from jax.experimental import pallas as pl
from jax.experimental.pallas import tpu as pltpu
from jax.experimental.pallas import tpu_sc as plsc

N = 10000
E = 160000
NPAD = 10240
NC, NS, L = 2, 16, 16
NW = NC * NS
T = 5120
EPAD = NW * T
CK = 128
NCH = T // CK
GPC = CK // L
RPT = NPAD // NS
SQRT3 = 3.0 ** 0.5
F32 = jnp.float32
I32 = jnp.int32

_MESH = plsc.VectorSubcoreMesh(
    core_axis_name="c", subcore_axis_name="s", num_cores=NC, num_subcores=NS)


def _rsqrt(n2):
    i = plsc.bitcast(n2, I32)
    i = jnp.full((L,), 0x5F3759DF, I32) - lax.shift_right_logical(
        i, jnp.full((L,), 1, I32))
    y = plsc.bitcast(i, F32)
    for _ in range(3):
        y = y * (1.5 - 0.5 * n2 * y * y)
    return y


def _geom_body(pos_hbm, src_hbm, dst_hbm, ln_hbm, ux_hbm, uy_hbm, uz_hbm,
               pos_v, src_v, dst_v, ln_v, ux_v, uy_v, uz_v):
    cid = lax.axis_index("c")
    sid = lax.axis_index("s")
    wid = sid * NC + cid
    base = wid * T
    pltpu.sync_copy(pos_hbm, pos_v)
    pltpu.sync_copy(src_hbm.at[pl.ds(base, T)], src_v)
    pltpu.sync_copy(dst_hbm.at[pl.ds(base, T)], dst_v)

    @pl.loop(0, T // L)
    def _groups(i):
        off = i * L
        s3 = src_v[pl.ds(off, L)] * 3
        d3 = dst_v[pl.ds(off, L)] * 3
        xs = plsc.load_gather(pos_v, [s3])
        ys = plsc.load_gather(pos_v, [s3 + 1])
        zs = plsc.load_gather(pos_v, [s3 + 2])
        xd = plsc.load_gather(pos_v, [d3])
        yd = plsc.load_gather(pos_v, [d3 + 1])
        zd = plsc.load_gather(pos_v, [d3 + 2])
        dx = xd - xs
        dy = yd - ys
        dz = zd - zs
        n2 = dx * dx + dy * dy + dz * dz + 1e-12
        y = _rsqrt(n2)
        ln_v[pl.ds(off, L)] = n2 * y
        f = y * SQRT3
        ux_v[pl.ds(off, L)] = dx * f
        uy_v[pl.ds(off, L)] = dy * f
        uz_v[pl.ds(off, L)] = dz * f

    pltpu.sync_copy(ln_v, ln_hbm.at[pl.ds(base, T)])
    pltpu.sync_copy(ux_v, ux_hbm.at[pl.ds(base, T)])
    pltpu.sync_copy(uy_v, uy_hbm.at[pl.ds(base, T)])
    pltpu.sync_copy(uz_v, uz_hbm.at[pl.ds(base, T)])


_SC_PARAMS = pltpu.CompilerParams(needs_layout_passes=False,
                                  use_tc_tiling_on_sc=False)

_sc_geom = pl.kernel(
    _geom_body,
    out_type=tuple(jax.ShapeDtypeStruct((EPAD,), F32) for _ in range(4)),
    mesh=_MESH,
    compiler_params=_SC_PARAMS,
    scratch_types=[
        pltpu.VMEM((NPAD * 3,), F32),
        pltpu.VMEM((T,), I32),
        pltpu.VMEM((T,), I32),
        pltpu.VMEM((T,), F32),
        pltpu.VMEM((T,), F32),
        pltpu.VMEM((T,), F32),
        pltpu.VMEM((T,), F32),
    ],
)


def _main_body(src3_hbm, dst3_hbm, nf_hbm, rad_hbm, ux_hbm, uy_hbm, uz_hbm,
               zeros_hbm, out_hbm, src2_v, dst2_v, ux_v, uy_v, uz_v, nf_v,
               rad_v, msg_v, agg_sh):
    cid = lax.axis_index("c")
    sid = lax.axis_index("s")
    wid = sid * NC + cid
    base = wid * T
    pltpu.sync_copy(src3_hbm.at[wid], src2_v)
    pltpu.sync_copy(dst3_hbm.at[wid], dst2_v)
    pltpu.sync_copy(ux_hbm.at[pl.ds(base, T)], ux_v)
    pltpu.sync_copy(uy_hbm.at[pl.ds(base, T)], uy_v)
    pltpu.sync_copy(uz_hbm.at[pl.ds(base, T)], uz_v)
    pltpu.sync_copy(zeros_hbm.at[pl.ds(sid * RPT, RPT)],
                    agg_sh.at[pl.ds(sid * RPT, RPT)])
    plsc.subcore_barrier()

    iota = lax.iota(I32, L)

    @pl.loop(0, NCH)
    def _chunks(j):
        pltpu.sync_copy(nf_hbm.at[src2_v.at[j]], nf_v)
        pltpu.sync_copy(rad_hbm.at[pl.ds(base + j * CK, CK)], rad_v)
        for g in range(GPC):
            rows = iota + (g * L)
            gate = jnp.zeros((L,), F32)
            for k in range(32):
                ck = jnp.bitwise_and(iota + k, 31)
                m = (plsc.load_gather(nf_v, [rows, ck])
                     * plsc.load_gather(rad_v, [rows, ck]))
                gate = gate + m
                plsc.store_scatter(msg_v, [rows, ck], m)
            uoff = j * CK + g * L
            plsc.store_scatter(msg_v, [rows, jnp.full((L,), 32, I32)],
                               ux_v[pl.ds(uoff, L)] * gate)
            plsc.store_scatter(msg_v, [rows, jnp.full((L,), 33, I32)],
                               uy_v[pl.ds(uoff, L)] * gate)
            plsc.store_scatter(msg_v, [rows, jnp.full((L,), 34, I32)],
                               uz_v[pl.ds(uoff, L)] * gate)
        pltpu.sync_copy(msg_v, agg_sh.at[dst2_v.at[j]], add=True)

    plsc.subcore_barrier()
    pltpu.sync_copy(agg_sh.at[pl.ds(sid * RPT, RPT)],
                    out_hbm.at[cid, pl.ds(sid * RPT, RPT)])


_sc_main = pl.kernel(
    _main_body,
    out_type=jax.ShapeDtypeStruct((NC, NPAD, 48), F32),
    mesh=_MESH,
    compiler_params=_SC_PARAMS,
    scratch_types=[
        pltpu.VMEM((NCH, CK), I32),
        pltpu.VMEM((NCH, CK), I32),
        pltpu.VMEM((T,), F32),
        pltpu.VMEM((T,), F32),
        pltpu.VMEM((T,), F32),
        pltpu.VMEM((CK, 32), F32),
        pltpu.VMEM((CK, 32), F32),
        pltpu.VMEM((CK, 48), F32),
        pltpu.VMEM_SHARED((NPAD, 48), F32),
    ],
)


def _embed_krn(x_ref, w_ref, b_ref, o_ref):
    h = jnp.dot(x_ref[...], w_ref[...],
                preferred_element_type=F32) + b_ref[...]
    o_ref[0:N, :] = h
    o_ref[N:NPAD, :] = jnp.zeros((NPAD - N, 32), F32)


_tc_embed = pl.pallas_call(
    _embed_krn, out_shape=jax.ShapeDtypeStruct((NPAD, 32), F32))

_RB = 4096


def _radial_krn(ln_ref, w1_ref, b1_ref, w2_ref, b2_ref, o_ref):
    x = ln_ref[...]
    t = x * w1_ref[...] + b1_ref[...]
    h = t * jax.nn.sigmoid(t)
    o_ref[...] = jnp.dot(h, w2_ref[...],
                         preferred_element_type=F32) + b2_ref[...]


_tc_radial = pl.pallas_call(
    _radial_krn,
    grid=(EPAD // _RB,),
    in_specs=[
        pl.BlockSpec((_RB, 1), lambda i: (i, 0)),
        pl.BlockSpec((1, 16), lambda i: (0, 0)),
        pl.BlockSpec((1, 16), lambda i: (0, 0)),
        pl.BlockSpec((16, 32), lambda i: (0, 0)),
        pl.BlockSpec((1, 32), lambda i: (0, 0)),
    ],
    out_specs=pl.BlockSpec((_RB, 32), lambda i: (i, 0)),
    out_shape=jax.ShapeDtypeStruct((EPAD, 32), F32),
)


def _dense0_krn(parts_ref, pos_ref, wp_ref, w1_ref, b1_ref, w2_ref, b2_ref,
                nf_ref, pos1_ref):
    p = (parts_ref[0] + parts_ref[1]) * 0.0625
    nf = jnp.dot(p[0:N, 0:32], wp_ref[...], preferred_element_type=F32)
    t = jnp.dot(nf, w1_ref[...], preferred_element_type=F32) + b1_ref[...]
    t = t * jax.nn.sigmoid(t)
    ro3 = jnp.dot(t, w2_ref[...], preferred_element_type=F32) + b2_ref[...]
    mbv = ro3 + p[0:N, 32:35]
    nf_ref[0:N, :] = nf
    nf_ref[N:NPAD, :] = jnp.zeros((NPAD - N, 32), F32)
    pos1_ref[0:N, :] = pos_ref[0:N, :] + mbv
    pos1_ref[N:NPAD, :] = jnp.zeros((NPAD - N, 3), F32)


_tc_dense0 = pl.pallas_call(
    _dense0_krn,
    out_shape=(jax.ShapeDtypeStruct((NPAD, 32), F32),
               jax.ShapeDtypeStruct((NPAD, 3), F32)))


def _dense1_krn(parts_ref, pos_ref, pos0_ref, wp_ref, w1_ref, b1_ref,
                w2_ref, b2_ref, o_ref):
    p = (parts_ref[0] + parts_ref[1]) * 0.0625
    nf = jnp.dot(p[0:N, 0:32], wp_ref[...], preferred_element_type=F32)
    t = jnp.dot(nf, w1_ref[...], preferred_element_type=F32) + b1_ref[...]
    t = t * jax.nn.sigmoid(t)
    ro3 = jnp.dot(t, w2_ref[...], preferred_element_type=F32) + b2_ref[...]
    pos2 = pos_ref[0:N, :] + ro3 + p[0:N, 32:35]
    o_ref[...] = (pos2 - jnp.mean(pos2, axis=0, keepdims=True)
                  - pos0_ref[...])


_tc_dense1 = pl.pallas_call(
    _dense1_krn, out_shape=jax.ShapeDtypeStruct((N, 3), F32))


def kernel(positions, node_attrs, time_embedding, edge_index, params):
    positions = positions.astype(F32)
    src = edge_index[0].astype(I32)
    dst = edge_index[1].astype(I32)
    pad = EPAD - E
    src_p = jnp.concatenate([src, jnp.zeros((pad,), I32)])
    dst_p = jnp.concatenate([dst, jnp.full((pad,), N, I32)])
    src3 = src_p.reshape(NW, NCH, CK)
    dst3 = dst_p.reshape(NW, NCH, CK)
    X = jnp.concatenate([node_attrs.astype(F32),
                         time_embedding.astype(F32)], axis=1)
    zeros_agg = jnp.zeros((NPAD, 48), F32)
    pos_pad0 = jnp.concatenate(
        [positions, jnp.zeros((NPAD - N, 3), F32)], axis=0)

    def row(v):
        return v.reshape(1, -1).astype(F32)

    nf_pad = _tc_embed(X, params['W_embed'].astype(F32),
                       row(params['b_embed']))
    pos_pad = pos_pad0
    for l in range(2):
        ln, ux, uy, uz = _sc_geom(pos_pad.reshape(-1), src_p, dst_p)
        rad = _tc_radial(ln.reshape(EPAD, 1),
                         params['Wr1_%d' % l].astype(F32),
                         row(params['br1_%d' % l]),
                         params['Wr2_%d' % l].astype(F32),
                         row(params['br2_%d' % l]))
        parts = _sc_main(src3, dst3, nf_pad, rad, ux, uy, uz, zeros_agg)
        wp = params['Wp_%d' % l].astype(F32)
        w1 = params['Wro1_%d' % l].astype(F32)
        b1 = row(params['bro1_%d' % l])
        w2 = params['Wro2_%d' % l][:, 32:35].astype(F32)
        b2 = row(params['bro2_%d' % l][32:35])
        if l == 0:
            nf_pad, pos_pad = _tc_dense0(parts, pos_pad, wp, w1, b1, w2, b2)
        else:
            out = _tc_dense1(parts, pos_pad, positions, wp, w1, b1, w2, b2)
    return out


# speedup vs baseline: 1.0932x; 1.0932x over previous
import jax
import jax.numpy as jnp
from jax import lax
from jax.experimental import pallas as pl
from jax.experimental.pallas import tpu as pltpu
from jax.experimental.pallas import tpu_sc as plsc

N = 10000
E = 160000
NPAD = 10240
NC, NS, L = 2, 16, 16
NW = NC * NS
T = 5120
EPAD = NW * T
CK = 128
NCH = T // CK
GPC = CK // L
RPT = NPAD // NS
SQRT3 = 3.0 ** 0.5
F32 = jnp.float32
I32 = jnp.int32

_MESH = plsc.VectorSubcoreMesh(
    core_axis_name="c", subcore_axis_name="s", num_cores=NC, num_subcores=NS)
_SC_PARAMS = pltpu.CompilerParams(needs_layout_passes=False,
                                  use_tc_tiling_on_sc=False,
                                  disable_bounds_checks=True)


def _rsqrt(n2):
    i = plsc.bitcast(n2, I32)
    i = jnp.full((L,), 0x5F3759DF, I32) - lax.shift_right_logical(
        i, jnp.full((L,), 1, I32))
    y = plsc.bitcast(i, F32)
    for _ in range(3):
        y = y * (1.5 - 0.5 * n2 * y * y)
    return y


def _edge_geom(src2_v, dst2_v, pos_v, j, g):
    s3 = src2_v[j, pl.ds(g * L, L)] * 3
    d3 = dst2_v[j, pl.ds(g * L, L)] * 3
    dx = plsc.load_gather(pos_v, [d3]) - plsc.load_gather(pos_v, [s3])
    dy = plsc.load_gather(pos_v, [d3 + 1]) - plsc.load_gather(pos_v, [s3 + 1])
    dz = plsc.load_gather(pos_v, [d3 + 2]) - plsc.load_gather(pos_v, [s3 + 2])
    n2 = dx * dx + dy * dy + dz * dz + 1e-12
    y = _rsqrt(n2)
    return dx, dy, dz, n2, y


def _geom_body(src3_hbm, dst3_hbm, pos_hbm, ln_hbm, src2_v, dst2_v, pos_v,
               ln_v):
    cid = lax.axis_index("c")
    sid = lax.axis_index("s")
    wid = sid * NC + cid
    base = wid * T
    pltpu.sync_copy(src3_hbm.at[wid], src2_v)
    pltpu.sync_copy(dst3_hbm.at[wid], dst2_v)
    pltpu.sync_copy(pos_hbm, pos_v)

    @pl.loop(0, NCH)
    def _chunks(j):
        @pl.loop(0, GPC)
        def _groups(g):
            _, _, _, n2, y = _edge_geom(src2_v, dst2_v, pos_v, j, g)
            ln_v[pl.ds(j * CK + g * L, L)] = n2 * y

    pltpu.sync_copy(ln_v, ln_hbm.at[pl.ds(base, T)])


_sc_geom = pl.kernel(
    _geom_body,
    out_type=jax.ShapeDtypeStruct((EPAD,), F32),
    mesh=_MESH,
    compiler_params=_SC_PARAMS,
    scratch_types=[
        pltpu.VMEM((NCH, CK), I32),
        pltpu.VMEM((NCH, CK), I32),
        pltpu.VMEM((NPAD * 3,), F32),
        pltpu.VMEM((T,), F32),
    ],
)


def _main_body(src3_hbm, dst3_hbm, pos_hbm, nf_hbm, rad_hbm, zeros_hbm,
               out_hbm, src2_v, dst2_v, pos_v, nf_v, rad_v, msg_v, agg_sh):
    cid = lax.axis_index("c")
    sid = lax.axis_index("s")
    wid = sid * NC + cid
    base = wid * T
    pltpu.sync_copy(src3_hbm.at[wid], src2_v)
    pltpu.sync_copy(dst3_hbm.at[wid], dst2_v)
    pltpu.sync_copy(pos_hbm, pos_v)
    pltpu.sync_copy(zeros_hbm.at[pl.ds(sid * RPT, RPT)],
                    agg_sh.at[pl.ds(sid * RPT, RPT)])
    plsc.subcore_barrier()

    iota = lax.iota(I32, L)

    @pl.loop(0, NCH)
    def _chunks(j):
        pltpu.sync_copy(nf_hbm.at[src2_v.at[j]], nf_v)
        pltpu.sync_copy(rad_hbm.at[pl.ds(base + j * CK, CK)], rad_v)

        @pl.loop(0, GPC)
        def _groups(g):
            dx, dy, dz, _, y = _edge_geom(src2_v, dst2_v, pos_v, j, g)
            f = y * SQRT3
            ux = dx * f
            uy = dy * f
            uz = dz * f
            rows = iota + (g * L)
            gate = jnp.zeros((L,), F32)
            for k in range(32):
                ck = jnp.bitwise_and(iota + k, 31)
                m = (plsc.load_gather(nf_v, [rows, ck])
                     * plsc.load_gather(rad_v, [rows, ck]))
                gate = gate + m
                plsc.store_scatter(msg_v, [rows, ck], m)
            plsc.store_scatter(msg_v, [rows, jnp.full((L,), 32, I32)],
                               ux * gate)
            plsc.store_scatter(msg_v, [rows, jnp.full((L,), 33, I32)],
                               uy * gate)
            plsc.store_scatter(msg_v, [rows, jnp.full((L,), 34, I32)],
                               uz * gate)
        pltpu.sync_copy(msg_v, agg_sh.at[dst2_v.at[j]], add=True)

    plsc.subcore_barrier()
    pltpu.sync_copy(agg_sh.at[pl.ds(sid * RPT, RPT)],
                    out_hbm.at[cid, pl.ds(sid * RPT, RPT)])


_sc_main = pl.kernel(
    _main_body,
    out_type=jax.ShapeDtypeStruct((NC, NPAD, 48), F32),
    mesh=_MESH,
    compiler_params=_SC_PARAMS,
    scratch_types=[
        pltpu.VMEM((NCH, CK), I32),
        pltpu.VMEM((NCH, CK), I32),
        pltpu.VMEM((NPAD * 3,), F32),
        pltpu.VMEM((CK, 32), F32),
        pltpu.VMEM((CK, 32), F32),
        pltpu.VMEM((CK, 48), F32),
        pltpu.VMEM_SHARED((NPAD, 48), F32),
    ],
)

_RB = 8192


def _radial_blk(ln_ref, w1_ref, b1_ref, w2_ref, b2_ref):
    t = ln_ref[...] * w1_ref[...] + b1_ref[...]
    h = t * jax.nn.sigmoid(t)
    return jnp.dot(h, w2_ref[...], preferred_element_type=F32) + b2_ref[...]


def _prep_krn(ln_ref, x_ref, we_ref, be_ref, w1_ref, b1_ref, w2_ref,
              b2_ref, rad_ref, h_ref):
    i = pl.program_id(0)
    rad_ref[...] = _radial_blk(ln_ref, w1_ref, b1_ref, w2_ref, b2_ref)

    @pl.when(i == 0)
    def _():
        h = jnp.dot(x_ref[...], we_ref[...],
                    preferred_element_type=F32) + be_ref[...]
        h_ref[0:N, :] = h
        h_ref[N:NPAD, :] = jnp.zeros((NPAD - N, 32), F32)


_w_spec = [
    pl.BlockSpec((1, 16), lambda i: (0, 0)),
    pl.BlockSpec((1, 16), lambda i: (0, 0)),
    pl.BlockSpec((16, 32), lambda i: (0, 0)),
    pl.BlockSpec((1, 32), lambda i: (0, 0)),
]

_tc_prep = pl.pallas_call(
    _prep_krn,
    grid=(EPAD // _RB,),
    in_specs=[pl.BlockSpec((_RB, 1), lambda i: (i, 0)),
              pl.BlockSpec((N, 10), lambda i: (0, 0)),
              pl.BlockSpec((10, 32), lambda i: (0, 0)),
              pl.BlockSpec((1, 32), lambda i: (0, 0))] + _w_spec,
    out_specs=(pl.BlockSpec((_RB, 32), lambda i: (i, 0)),
               pl.BlockSpec((NPAD, 32), lambda i: (0, 0))),
    out_shape=(jax.ShapeDtypeStruct((EPAD, 32), F32),
               jax.ShapeDtypeStruct((NPAD, 32), F32)))


def _radial_krn(ln_ref, w1_ref, b1_ref, w2_ref, b2_ref, rad_ref):
    rad_ref[...] = _radial_blk(ln_ref, w1_ref, b1_ref, w2_ref, b2_ref)


_tc_radial = pl.pallas_call(
    _radial_krn,
    grid=(EPAD // _RB,),
    in_specs=[pl.BlockSpec((_RB, 1), lambda i: (i, 0))] + _w_spec,
    out_specs=pl.BlockSpec((_RB, 32), lambda i: (i, 0)),
    out_shape=jax.ShapeDtypeStruct((EPAD, 32), F32))


def _dense0_krn(parts_ref, pos_ref, wp_ref, w1_ref, b1_ref, w2_ref, b2_ref,
                nf_ref, pos1_ref):
    p = (parts_ref[0] + parts_ref[1]) * 0.0625
    nf = jnp.dot(p[0:N, 0:32], wp_ref[...], preferred_element_type=F32)
    t = jnp.dot(nf, w1_ref[...], preferred_element_type=F32) + b1_ref[...]
    t = t * jax.nn.sigmoid(t)
    ro3 = jnp.dot(t, w2_ref[...], preferred_element_type=F32) + b2_ref[...]
    mbv = ro3 + p[0:N, 32:35]
    nf_ref[0:N, :] = nf
    nf_ref[N:NPAD, :] = jnp.zeros((NPAD - N, 32), F32)
    pos1_ref[0:N, :] = pos_ref[0:N, :] + mbv
    pos1_ref[N:NPAD, :] = jnp.zeros((NPAD - N, 3), F32)


_tc_dense0 = pl.pallas_call(
    _dense0_krn,
    out_shape=(jax.ShapeDtypeStruct((NPAD, 32), F32),
               jax.ShapeDtypeStruct((NPAD, 3), F32)))


def _dense1_krn(parts_ref, pos_ref, pos0_ref, wp_ref, w1_ref, b1_ref,
                w2_ref, b2_ref, o_ref):
    p = (parts_ref[0] + parts_ref[1]) * 0.0625
    nf = jnp.dot(p[0:N, 0:32], wp_ref[...], preferred_element_type=F32)
    t = jnp.dot(nf, w1_ref[...], preferred_element_type=F32) + b1_ref[...]
    t = t * jax.nn.sigmoid(t)
    ro3 = jnp.dot(t, w2_ref[...], preferred_element_type=F32) + b2_ref[...]
    pos2 = pos_ref[0:N, :] + ro3 + p[0:N, 32:35]
    o_ref[...] = (pos2 - jnp.mean(pos2, axis=0, keepdims=True)
                  - pos0_ref[...])


_tc_dense1 = pl.pallas_call(
    _dense1_krn, out_shape=jax.ShapeDtypeStruct((N, 3), F32))


def kernel(positions, node_attrs, time_embedding, edge_index, params):
    positions = positions.astype(F32)
    src = edge_index[0].astype(I32)
    dst = edge_index[1].astype(I32)
    pad = EPAD - E
    src_p = jnp.concatenate([src, jnp.zeros((pad,), I32)])
    dst_p = jnp.concatenate([dst, jnp.full((pad,), N, I32)])
    src3 = src_p.reshape(NW, NCH, CK)
    dst3 = dst_p.reshape(NW, NCH, CK)
    X = jnp.concatenate([node_attrs.astype(F32),
                         time_embedding.astype(F32)], axis=1)
    zeros_agg = jnp.zeros((NPAD, 48), F32)
    pos_pad = jnp.concatenate(
        [positions, jnp.zeros((NPAD - N, 3), F32)], axis=0)

    def row(v):
        return v.reshape(1, -1).astype(F32)

    nf_pad = None
    for l in range(2):
        ln = _sc_geom(src3, dst3, pos_pad.reshape(-1))
        rw = (ln.reshape(EPAD, 1), params['Wr1_%d' % l].astype(F32),
              row(params['br1_%d' % l]), params['Wr2_%d' % l].astype(F32),
              row(params['br2_%d' % l]))
        if l == 0:
            rad, nf_pad = _tc_prep(rw[0], X, params['W_embed'].astype(F32),
                                   row(params['b_embed']), *rw[1:])
        else:
            rad = _tc_radial(*rw)
        parts = _sc_main(src3, dst3, pos_pad.reshape(-1), nf_pad, rad,
                         zeros_agg)
        wp = params['Wp_%d' % l].astype(F32)
        w1 = params['Wro1_%d' % l].astype(F32)
        b1 = row(params['bro1_%d' % l])
        w2 = params['Wro2_%d' % l][:, 32:35].astype(F32)
        b2 = row(params['bro2_%d' % l][32:35])
        if l == 0:
            nf_pad, pos_pad = _tc_dense0(parts, pos_pad, wp, w1, b1, w2, b2)
        else:
            out = _tc_dense1(parts, pos_pad, positions, wp, w1, b1, w2, b2)
    return out


# speedup vs baseline: 1.3559x; 1.2403x over previous
import jax
import jax.numpy as jnp
from jax import lax
from jax.experimental import pallas as pl
from jax.experimental.pallas import tpu as pltpu
from jax.experimental.pallas import tpu_sc as plsc

N = 10000
E = 160000
NPAD = 10240
NC, NS, L = 2, 16, 16
NW = NC * NS
T = 5120
EPAD = NW * T
CK = 128
NCH = T // CK
GPC = CK // L
RPT = NPAD // NS
SQRT3 = 3.0 ** 0.5
F32 = jnp.float32
I32 = jnp.int32

_MESH = plsc.VectorSubcoreMesh(
    core_axis_name="c", subcore_axis_name="s", num_cores=NC, num_subcores=NS)
_SC_PARAMS = pltpu.CompilerParams(needs_layout_passes=False,
                                  use_tc_tiling_on_sc=False,
                                  disable_bounds_checks=True)


def _rsqrt(n2):
    i = plsc.bitcast(n2, I32)
    i = jnp.full((L,), 0x5F3759DF, I32) - lax.shift_right_logical(
        i, jnp.full((L,), 1, I32))
    y = plsc.bitcast(i, F32)
    for _ in range(3):
        y = y * (1.5 - 0.5 * n2 * y * y)
    return y


def _edge_geom(src2_v, dst2_v, pos_v, j, g):
    s3 = src2_v[j, pl.ds(g * L, L)] * 3
    d3 = dst2_v[j, pl.ds(g * L, L)] * 3
    dx = plsc.load_gather(pos_v, [d3]) - plsc.load_gather(pos_v, [s3])
    dy = plsc.load_gather(pos_v, [d3 + 1]) - plsc.load_gather(pos_v, [s3 + 1])
    dz = plsc.load_gather(pos_v, [d3 + 2]) - plsc.load_gather(pos_v, [s3 + 2])
    n2 = dx * dx + dy * dy + dz * dz + 1e-12
    y = _rsqrt(n2)
    return dx, dy, dz, n2, y


def _geom_body(src3_hbm, dst3_hbm, pos_hbm, ln_hbm, src2_v, dst2_v, pos_v,
               ln_v):
    cid = lax.axis_index("c")
    sid = lax.axis_index("s")
    wid = sid * NC + cid
    base = wid * T
    pltpu.sync_copy(src3_hbm.at[wid], src2_v)
    pltpu.sync_copy(dst3_hbm.at[wid], dst2_v)
    pltpu.sync_copy(pos_hbm, pos_v)

    @pl.loop(0, NCH)
    def _chunks(j):
        @pl.loop(0, GPC)
        def _groups(g):
            _, _, _, n2, y = _edge_geom(src2_v, dst2_v, pos_v, j, g)
            ln_v[pl.ds(j * CK + g * L, L)] = n2 * y

    pltpu.sync_copy(ln_v, ln_hbm.at[pl.ds(base, T)])


_sc_geom = pl.kernel(
    _geom_body,
    out_type=jax.ShapeDtypeStruct((EPAD,), F32),
    mesh=_MESH,
    compiler_params=_SC_PARAMS,
    scratch_types=[
        pltpu.VMEM((NCH, CK), I32),
        pltpu.VMEM((NCH, CK), I32),
        pltpu.VMEM((NPAD * 3,), F32),
        pltpu.VMEM((T,), F32),
    ],
)


def _main_body(src3_hbm, dst3_hbm, pos_hbm, nf_hbm, rad_hbm, zeros_hbm,
               out_hbm, src2_v, dst2_v, pos_v, nf_v0, rad_v0, nf_v1,
               rad_v1, msg_v, sem0, sem1, agg_sh):
    cid = lax.axis_index("c")
    sid = lax.axis_index("s")
    wid = sid * NC + cid
    base = wid * T
    pltpu.sync_copy(src3_hbm.at[wid], src2_v)
    pltpu.sync_copy(dst3_hbm.at[wid], dst2_v)
    pltpu.sync_copy(pos_hbm, pos_v)
    pltpu.sync_copy(zeros_hbm.at[pl.ds(sid * RPT, RPT)],
                    agg_sh.at[pl.ds(sid * RPT, RPT)])
    plsc.subcore_barrier()

    iota = lax.iota(I32, L)
    bufs = ((nf_v0, rad_v0, sem0), (nf_v1, rad_v1, sem1))

    def _fire(j, nfb, rdb, sem):
        pltpu.async_copy(nf_hbm.at[src2_v.at[j]], nfb, sem)
        pltpu.async_copy(rad_hbm.at[pl.ds(base + j * CK, CK)], rdb, sem)

    def _drain(j, nfb, rdb, sem):
        pltpu.make_async_copy(nf_hbm.at[src2_v.at[j]], nfb, sem).wait()
        pltpu.make_async_copy(rad_hbm.at[pl.ds(base + j * CK, CK)], rdb,
                              sem).wait()

    _fire(0, *bufs[0])

    @pl.loop(0, NCH // 2)
    def _chunks(jj):
        for par in range(2):
            j = jj * 2 + par
            nfb, rdb, sem = bufs[par]
            _drain(j, nfb, rdb, sem)
            if par == 0:
                _fire(j + 1, *bufs[1])
            else:
                @pl.when(j + 1 < NCH)
                def _():
                    _fire(j + 1, *bufs[0])

            @pl.loop(0, GPC)
            def _groups(g):
                dx, dy, dz, _, y = _edge_geom(src2_v, dst2_v, pos_v, j, g)
                f = y * SQRT3
                ux = dx * f
                uy = dy * f
                uz = dz * f
                rows = iota + (g * L)
                gate = jnp.zeros((L,), F32)
                for k in range(32):
                    ck = jnp.bitwise_and(iota + k, 31)
                    m = (plsc.load_gather(nfb, [rows, ck])
                         * plsc.load_gather(rdb, [rows, ck]))
                    gate = gate + m
                    plsc.store_scatter(msg_v, [rows, ck], m)
                plsc.store_scatter(msg_v, [rows, jnp.full((L,), 32, I32)],
                                   ux * gate)
                plsc.store_scatter(msg_v, [rows, jnp.full((L,), 33, I32)],
                                   uy * gate)
                plsc.store_scatter(msg_v, [rows, jnp.full((L,), 34, I32)],
                                   uz * gate)
            pltpu.sync_copy(msg_v, agg_sh.at[dst2_v.at[j]], add=True)

    plsc.subcore_barrier()
    pltpu.sync_copy(agg_sh.at[pl.ds(sid * RPT, RPT)],
                    out_hbm.at[cid, pl.ds(sid * RPT, RPT)])


_sc_main = pl.kernel(
    _main_body,
    out_type=jax.ShapeDtypeStruct((NC, NPAD, 48), F32),
    mesh=_MESH,
    compiler_params=_SC_PARAMS,
    scratch_types=[
        pltpu.VMEM((NCH, CK), I32),
        pltpu.VMEM((NCH, CK), I32),
        pltpu.VMEM((NPAD * 3,), F32),
        pltpu.VMEM((CK, 32), F32),
        pltpu.VMEM((CK, 32), F32),
        pltpu.VMEM((CK, 32), F32),
        pltpu.VMEM((CK, 32), F32),
        pltpu.VMEM((CK, 48), F32),
        pltpu.SemaphoreType.DMA,
        pltpu.SemaphoreType.DMA,
        pltpu.VMEM_SHARED((NPAD, 48), F32),
    ],
)

_RB = 8192


def _radial_blk(ln_ref, w1_ref, b1_ref, w2_ref, b2_ref):
    t = ln_ref[...] * w1_ref[...] + b1_ref[...]
    h = t * jax.nn.sigmoid(t)
    return jnp.dot(h, w2_ref[...], preferred_element_type=F32) + b2_ref[...]


def _prep_krn(ln_ref, x_ref, we_ref, be_ref, w1_ref, b1_ref, w2_ref,
              b2_ref, rad_ref, h_ref):
    i = pl.program_id(0)
    rad_ref[...] = _radial_blk(ln_ref, w1_ref, b1_ref, w2_ref, b2_ref)

    @pl.when(i == 0)
    def _():
        h = jnp.dot(x_ref[...], we_ref[...],
                    preferred_element_type=F32) + be_ref[...]
        h_ref[0:N, :] = h
        h_ref[N:NPAD, :] = jnp.zeros((NPAD - N, 32), F32)


_w_spec = [
    pl.BlockSpec((1, 16), lambda i: (0, 0)),
    pl.BlockSpec((1, 16), lambda i: (0, 0)),
    pl.BlockSpec((16, 32), lambda i: (0, 0)),
    pl.BlockSpec((1, 32), lambda i: (0, 0)),
]

_tc_prep = pl.pallas_call(
    _prep_krn,
    grid=(EPAD // _RB,),
    in_specs=[pl.BlockSpec((_RB, 1), lambda i: (i, 0)),
              pl.BlockSpec((N, 10), lambda i: (0, 0)),
              pl.BlockSpec((10, 32), lambda i: (0, 0)),
              pl.BlockSpec((1, 32), lambda i: (0, 0))] + _w_spec,
    out_specs=(pl.BlockSpec((_RB, 32), lambda i: (i, 0)),
               pl.BlockSpec((NPAD, 32), lambda i: (0, 0))),
    out_shape=(jax.ShapeDtypeStruct((EPAD, 32), F32),
               jax.ShapeDtypeStruct((NPAD, 32), F32)))


def _radial_krn(ln_ref, w1_ref, b1_ref, w2_ref, b2_ref, rad_ref):
    rad_ref[...] = _radial_blk(ln_ref, w1_ref, b1_ref, w2_ref, b2_ref)


_tc_radial = pl.pallas_call(
    _radial_krn,
    grid=(EPAD // _RB,),
    in_specs=[pl.BlockSpec((_RB, 1), lambda i: (i, 0))] + _w_spec,
    out_specs=pl.BlockSpec((_RB, 32), lambda i: (i, 0)),
    out_shape=jax.ShapeDtypeStruct((EPAD, 32), F32))


def _dense0_krn(parts_ref, pos_ref, wp_ref, w1_ref, b1_ref, w2_ref, b2_ref,
                nf_ref, pos1_ref):
    p = (parts_ref[0] + parts_ref[1]) * 0.0625
    nf = jnp.dot(p[0:N, 0:32], wp_ref[...], preferred_element_type=F32)
    t = jnp.dot(nf, w1_ref[...], preferred_element_type=F32) + b1_ref[...]
    t = t * jax.nn.sigmoid(t)
    ro3 = jnp.dot(t, w2_ref[...], preferred_element_type=F32) + b2_ref[...]
    mbv = ro3 + p[0:N, 32:35]
    nf_ref[0:N, :] = nf
    nf_ref[N:NPAD, :] = jnp.zeros((NPAD - N, 32), F32)
    pos1_ref[0:N, :] = pos_ref[0:N, :] + mbv
    pos1_ref[N:NPAD, :] = jnp.zeros((NPAD - N, 3), F32)


_tc_dense0 = pl.pallas_call(
    _dense0_krn,
    out_shape=(jax.ShapeDtypeStruct((NPAD, 32), F32),
               jax.ShapeDtypeStruct((NPAD, 3), F32)))


def _dense1_krn(parts_ref, pos_ref, pos0_ref, wp_ref, w1_ref, b1_ref,
                w2_ref, b2_ref, o_ref):
    p = (parts_ref[0] + parts_ref[1]) * 0.0625
    nf = jnp.dot(p[0:N, 0:32], wp_ref[...], preferred_element_type=F32)
    t = jnp.dot(nf, w1_ref[...], preferred_element_type=F32) + b1_ref[...]
    t = t * jax.nn.sigmoid(t)
    ro3 = jnp.dot(t, w2_ref[...], preferred_element_type=F32) + b2_ref[...]
    pos2 = pos_ref[0:N, :] + ro3 + p[0:N, 32:35]
    o_ref[...] = (pos2 - jnp.mean(pos2, axis=0, keepdims=True)
                  - pos0_ref[...])


_tc_dense1 = pl.pallas_call(
    _dense1_krn, out_shape=jax.ShapeDtypeStruct((N, 3), F32))


def kernel(positions, node_attrs, time_embedding, edge_index, params):
    positions = positions.astype(F32)
    src = edge_index[0].astype(I32)
    dst = edge_index[1].astype(I32)
    pad = EPAD - E
    src_p = jnp.concatenate([src, jnp.zeros((pad,), I32)])
    dst_p = jnp.concatenate([dst, jnp.full((pad,), N, I32)])
    src3 = src_p.reshape(NW, NCH, CK)
    dst3 = dst_p.reshape(NW, NCH, CK)
    X = jnp.concatenate([node_attrs.astype(F32),
                         time_embedding.astype(F32)], axis=1)
    zeros_agg = jnp.zeros((NPAD, 48), F32)
    pos_pad = jnp.concatenate(
        [positions, jnp.zeros((NPAD - N, 3), F32)], axis=0)

    def row(v):
        return v.reshape(1, -1).astype(F32)

    nf_pad = None
    for l in range(2):
        ln = _sc_geom(src3, dst3, pos_pad.reshape(-1))
        rw = (ln.reshape(EPAD, 1), params['Wr1_%d' % l].astype(F32),
              row(params['br1_%d' % l]), params['Wr2_%d' % l].astype(F32),
              row(params['br2_%d' % l]))
        if l == 0:
            rad, nf_pad = _tc_prep(rw[0], X, params['W_embed'].astype(F32),
                                   row(params['b_embed']), *rw[1:])
        else:
            rad = _tc_radial(*rw)
        parts = _sc_main(src3, dst3, pos_pad.reshape(-1), nf_pad, rad,
                         zeros_agg)
        wp = params['Wp_%d' % l].astype(F32)
        w1 = params['Wro1_%d' % l].astype(F32)
        b1 = row(params['bro1_%d' % l])
        w2 = params['Wro2_%d' % l][:, 32:35].astype(F32)
        b2 = row(params['bro2_%d' % l][32:35])
        if l == 0:
            nf_pad, pos_pad = _tc_dense0(parts, pos_pad, wp, w1, b1, w2, b2)
        else:
            out = _tc_dense1(parts, pos_pad, positions, wp, w1, b1, w2, b2)
    return out
